# baseline (speedup 1.0000x reference)
import functools

import jax
import jax.numpy as jnp
from jax import lax
from jax.experimental import pallas as pl
from jax.experimental.pallas import tpu as pltpu, tpu_sc as plsc

_B, _N, _E, _FEAT = 32, 10000, 160000, 256
_C = _B * 9
_H = _C // 2
_NC, _NS = 2, 16
_EP = 163840
_TRASH = _N
_DROWS = 10240
_SROWS = 10112
_RPT = 632
_TN = 400
_KT = 3600
_EPS = 1e-5

_MESH = plsc.VectorSubcoreMesh(
    core_axis_name="c", subcore_axis_name="s", num_cores=_NC, num_subcores=_NS)
_SC_PARAMS = pltpu.CompilerParams(use_tc_tiling_on_sc=False)


def _fill_rows(ref, nrows, ncol16, value):
    def row(i, _):
        def col(k, _):
            ref[i, pl.ds(k * 16, 16)] = jnp.full((16,), value, ref.dtype)
            return 0
        return lax.fori_loop(0, ncol16, col, 0)
    lax.fori_loop(0, nrows, row, 0)


def _deg_body(dst_hbm, out_hbm, dst_v, ones_v, stage_v, acc, sem):
    c = lax.axis_index("c")
    s = lax.axis_index("s")
    w = c * _NS + s
    pltpu.sync_copy(dst_hbm.at[w], dst_v)
    _fill_rows(ones_v, 128, 1, 1.0)
    _fill_rows(stage_v, 640, 1, 0.0)
    pltpu.sync_copy(stage_v, acc.at[pl.ds(s * 640, 640)])
    plsc.subcore_barrier()

    def chunk(j, _):
        pltpu.sync_copy(ones_v, acc.at[dst_v.at[j]], add=True)
        return 0
    lax.fori_loop(0, 40, chunk, 0)
    plsc.subcore_barrier()
    pltpu.sync_copy(acc.at[pl.ds(s * 640, 640)], stage_v)
    pltpu.sync_copy(stage_v, out_hbm.at[c, pl.ds(s * 640, 640)])


_deg_call = functools.partial(
    pl.kernel,
    out_type=jax.ShapeDtypeStruct((_NC, _DROWS, 16), jnp.float32),
    mesh=_MESH,
    scratch_types=[
        pltpu.VMEM((40, 128), jnp.int32),
        pltpu.VMEM((128, 16), jnp.float32),
        pltpu.VMEM((640, 16), jnp.float32),
        pltpu.VMEM_SHARED((_DROWS, 16), jnp.float32),
        pltpu.SemaphoreType.DMA,
    ],
    compiler_params=_SC_PARAMS,
)


_NBUF = 1
_RCH = 512
_CPT = 20
_Q = 80


def _shift_src(src_v, off):
    def row(i, _):
        def col(k, _):
            sl = (i, pl.ds(k * 16, 16))
            src_v[sl] = src_v[sl] + off
            return 0
        return lax.fori_loop(0, 8, col, 0)
    lax.fori_loop(0, 80, row, 0)


def _seg_body(tab_hbm, src_hbm, dst_hbm, out_hbm, src_v, dst_v, rows_v, acc,
              *sems):
    c = lax.axis_index("c")
    s = lax.axis_index("s")
    sg, ss = sems[:_NBUF], sems[_NBUF:]
    pltpu.sync_copy(dst_hbm.at[s], dst_v)
    base = s * _RPT
    for p in range(2):
        q = c * 2 + p
        pltpu.sync_copy(src_hbm.at[q, s], src_v)
        _fill_rows(rows_v.at[0], 512, _Q // 16, 0.0)
        pltpu.sync_copy(rows_v.at[0],
                        acc.at[pl.ds(base, 512)])
        pltpu.sync_copy(rows_v.at[0, pl.ds(0, _RPT - 512)],
                        acc.at[pl.ds(base + 512, _RPT - 512)])
        plsc.subcore_barrier()

        for b in range(_NBUF):
            pltpu.async_copy(tab_hbm.at[src_v.at[b]], rows_v.at[b], sg[b])

        def wave(g, _):
            j0 = g * _NBUF
            for b in range(_NBUF):
                pltpu.make_async_copy(tab_hbm.at[src_v.at[j0 + b]],
                                      rows_v.at[b], sg[b]).wait()
                pltpu.async_copy(rows_v.at[b], acc.at[dst_v.at[j0 + b]],
                                 ss[b], add=True)
            for b in range(_NBUF):
                jn = jnp.minimum(j0 + _NBUF + b, _CPT - 1)
                pltpu.make_async_copy(rows_v.at[b], acc.at[dst_v.at[j0 + b]],
                                      ss[b]).wait()
                pltpu.async_copy(tab_hbm.at[src_v.at[jn]], rows_v.at[b],
                                 sg[b])
            return 0
        lax.fori_loop(0, _CPT // _NBUF, wave, 0)
        for b in range(_NBUF):
            pltpu.make_async_copy(tab_hbm.at[src_v.at[0]], rows_v.at[b],
                                  sg[b]).wait()
        plsc.subcore_barrier()

        @pl.when(s < _NS - 1)
        def _():
            pltpu.sync_copy(acc.at[pl.ds(base, _RPT)],
                            out_hbm.at[q, pl.ds(base, _RPT)])

        @pl.when(s == _NS - 1)
        def _():
            pltpu.sync_copy(acc.at[pl.ds(base, _N - 15 * _RPT)],
                            out_hbm.at[q, pl.ds(base, _N - 15 * _RPT)])

        if p == 0:
            plsc.subcore_barrier()


_seg_call = functools.partial(
    pl.kernel,
    out_type=jax.ShapeDtypeStruct((4, _N, _Q), jnp.float32),
    mesh=_MESH,
    scratch_types=[
        pltpu.VMEM((_CPT, _RCH), jnp.int32),
        pltpu.VMEM((_CPT, _RCH), jnp.int32),
        pltpu.VMEM((_NBUF, _RCH, _Q), jnp.float32),
        pltpu.VMEM_SHARED((_SROWS, _Q), jnp.float32),
    ] + [pltpu.SemaphoreType.DMA] * (2 * _NBUF),
    compiler_params=_SC_PARAMS,
)


def _layer_body(segp_ref, xsp_ref, dinv_ref, M_ref, A_ref, bb_ref,
                h_ref, hsp_ref):
    seg = jnp.concatenate([segp_ref[q][:, :72] for q in range(4)], axis=-1)
    xs = jnp.concatenate([xsp_ref[q][:, :72] for q in range(4)], axis=-1)
    y = (seg + xs) * dinv_ref[...]
    y = jnp.dot(y, M_ref[...], preferred_element_type=jnp.float32) + bb_ref[...]
    m = jnp.dot(y, A_ref[...], preferred_element_type=jnp.float32)
    d = y - m
    v = jnp.dot(d * d, A_ref[...], preferred_element_type=jnp.float32)
    h = jnp.maximum(d * jax.lax.rsqrt(v + _EPS), 0.0)
    h_ref[...] = h
    hs = h * dinv_ref[...]
    zpad = jnp.zeros((hs.shape[0], _Q - 72), jnp.float32)
    for q in range(4):
        hsp_ref[q] = jnp.concatenate([hs[:, q * 72:(q + 1) * 72], zpad],
                                     axis=-1)


def _layer_tc(segp, xsp, dinv, M, A, bb):
    grid = (_N // _TN,)
    packed = pl.BlockSpec((4, _TN, _Q), lambda i: (0, i, 0))
    row = pl.BlockSpec((_TN, _C), lambda i: (i, 0))
    rowscale = pl.BlockSpec((_TN, 1), lambda i: (i, 0))
    full = pl.BlockSpec((_C, _C), lambda i: (0, 0))
    vec = pl.BlockSpec((1, _C), lambda i: (0, 0))
    return pl.pallas_call(
        _layer_body,
        grid=grid,
        in_specs=[packed, packed, rowscale, full, full, vec],
        out_specs=[row, packed],
        out_shape=[jax.ShapeDtypeStruct((_N, _C), jnp.float32),
                   jax.ShapeDtypeStruct((4, _N, _Q), jnp.float32)],
    )(segp, xsp, dinv, M, A, bb)


def _heads_body(flat_ref, wmu_ref, wvar_ref, bmu_ref, bvar_ref,
                mu_ref, lv_ref, accmu, acclv):
    i = pl.program_id(0)

    @pl.when(i == 0)
    def _():
        accmu[...] = jnp.zeros_like(accmu)
        acclv[...] = jnp.zeros_like(acclv)

    f = flat_ref[0]
    accmu[...] += jnp.dot(f, wmu_ref[0], preferred_element_type=jnp.float32)
    acclv[...] += jnp.dot(f, wvar_ref[0], preferred_element_type=jnp.float32)

    @pl.when(i == (_N * 9) // _KT - 1)
    def _():
        mu_ref[...] = accmu[...] + bmu_ref[...]
        lv_ref[...] = acclv[...] + bvar_ref[...]


def _heads_tc(flat3, Wmu, bmu, Wvar, bvar):
    nk = (_N * 9) // _KT
    return pl.pallas_call(
        _heads_body,
        grid=(nk,),
        in_specs=[
            pl.BlockSpec((1, _B, _KT), lambda i: (i, 0, 0)),
            pl.BlockSpec((1, _KT, _FEAT), lambda i: (i, 0, 0)),
            pl.BlockSpec((1, _KT, _FEAT), lambda i: (i, 0, 0)),
            pl.BlockSpec((1, _FEAT), lambda i: (0, 0)),
            pl.BlockSpec((1, _FEAT), lambda i: (0, 0)),
        ],
        out_specs=[pl.BlockSpec((_B, _FEAT), lambda i: (0, 0))] * 2,
        out_shape=[jax.ShapeDtypeStruct((_B, _FEAT), jnp.float32)] * 2,
        scratch_shapes=[pltpu.VMEM((_B, _FEAT), jnp.float32)] * 2,
    )(flat3, Wmu.reshape(nk, _KT, _FEAT), Wvar.reshape(nk, _KT, _FEAT),
      bmu.reshape(1, _FEAT), bvar.reshape(1, _FEAT))


def kernel(featurein, edge_index, W1, b1, W2, b2, W3, b3, Wmu, bmu, Wvar, bvar):
    src = edge_index[0]
    dst = edge_index[1]

    npad = _EP - _E
    srcp = jnp.concatenate([src, jnp.zeros((npad,), jnp.int32)])
    dstp = jnp.concatenate([dst, jnp.full((npad,), _TRASH, jnp.int32)])
    src_seg = (srcp[None, :] + (jnp.arange(4, dtype=jnp.int32) * _N)[:, None]
               ).reshape(4, _NS, _CPT, _RCH)
    dst_seg = dstp.reshape(_NS, _CPT, _RCH)
    dst_deg = dstp.reshape(_NC * _NS, 40, 128)

    degp = _deg_call(_deg_body)(dst_deg)
    deg = degp[0, :_N, 0] + degp[1, :_N, 0] + 1.0
    dinv = jax.lax.rsqrt(deg).reshape(_N, 1)

    x_t = featurein.transpose(1, 0, 2).reshape(_N, _C)
    hsp = (x_t * dinv).reshape(_N, 4, 72).transpose(1, 0, 2)
    hsp = jnp.pad(hsp, ((0, 0), (0, 0), (0, _Q - 72)))

    eye = jnp.eye(_B, dtype=jnp.float32)
    A = jnp.kron(eye, jnp.full((9, 9), 1.0 / 9.0, jnp.float32))
    seg_fn = _seg_call(_seg_body)
    h = None
    for (W, b) in ((W1, b1), (W2, b2), (W3, b3)):
        M = jnp.kron(eye, W)
        bb = jnp.tile(b, (_B,)).reshape(1, _C)
        segp = seg_fn(hsp.reshape(4 * _N, _Q), src_seg, dst_seg)
        h, hsp = _layer_tc(segp, hsp, dinv, M, A, bb)

    nk = (_N * 9) // _KT
    flat3 = h.reshape(nk, _KT // 9, _B, 9).transpose(0, 2, 1, 3)
    flat3 = flat3.reshape(nk, _B, _KT)
    mu, logvar = _heads_tc(flat3, Wmu, bmu, Wvar, bvar)
    return (mu, logvar)


# speedup vs baseline: 1.2235x; 1.2235x over previous
import functools

import jax
import jax.numpy as jnp
from jax import lax
from jax.experimental import pallas as pl
from jax.experimental.pallas import tpu as pltpu, tpu_sc as plsc

_B, _N, _E, _FEAT = 32, 10000, 160000, 256
_C = _B * 9
_H = _C // 2
_NC, _NS = 2, 16
_EP = 163840
_TRASH = _N
_DROWS = 10240
_SROWS = 10112
_RPT = 632
_TN = 400
_EPS = 1e-5

_MESH = plsc.VectorSubcoreMesh(
    core_axis_name="c", subcore_axis_name="s", num_cores=_NC, num_subcores=_NS)
_SC_PARAMS = pltpu.CompilerParams(use_tc_tiling_on_sc=False)


def _fill_rows(ref, nrows, ncol16, value):
    def row(i, _):
        def col(k, _):
            ref[i, pl.ds(k * 16, 16)] = jnp.full((16,), value, ref.dtype)
            return 0
        return lax.fori_loop(0, ncol16, col, 0)
    lax.fori_loop(0, nrows, row, 0)


def _deg_body(dst_hbm, out_hbm, dst_v, ones_v, stage_v, acc, sem):
    c = lax.axis_index("c")
    s = lax.axis_index("s")
    w = c * _NS + s
    pltpu.sync_copy(dst_hbm.at[w], dst_v)
    _fill_rows(ones_v, 128, 1, 1.0)
    _fill_rows(stage_v, 640, 1, 0.0)
    pltpu.sync_copy(stage_v, acc.at[pl.ds(s * 640, 640)])
    plsc.subcore_barrier()

    def chunk(j, _):
        pltpu.sync_copy(ones_v, acc.at[dst_v.at[j]], add=True)
        return 0
    lax.fori_loop(0, 40, chunk, 0)
    plsc.subcore_barrier()
    pltpu.sync_copy(acc.at[pl.ds(s * 640, 640)], stage_v)
    pltpu.sync_copy(stage_v, out_hbm.at[c, pl.ds(s * 640, 640)])


_deg_call = functools.partial(
    pl.kernel,
    out_type=jax.ShapeDtypeStruct((_NC, _DROWS, 16), jnp.float32),
    mesh=_MESH,
    scratch_types=[
        pltpu.VMEM((40, 128), jnp.int32),
        pltpu.VMEM((128, 16), jnp.float32),
        pltpu.VMEM((640, 16), jnp.float32),
        pltpu.VMEM_SHARED((_DROWS, 16), jnp.float32),
        pltpu.SemaphoreType.DMA,
    ],
    compiler_params=_SC_PARAMS,
)


def _seg_body(tab_hbm, src_hbm, dst_hbm, out_hbm, src_v, dst_v, rows_v, acc,
              sem):
    c = lax.axis_index("c")
    s = lax.axis_index("s")
    pltpu.sync_copy(src_hbm.at[c, s], src_v)
    pltpu.sync_copy(dst_hbm.at[s], dst_v)
    _fill_rows(rows_v, 128, 9, 0.0)
    base = s * _RPT
    for p in range(4):
        pltpu.sync_copy(rows_v, acc.at[pl.ds(base + p * 128, 128)])
    pltpu.sync_copy(rows_v.at[pl.ds(0, _RPT - 512)],
                    acc.at[pl.ds(base + 512, _RPT - 512)])
    plsc.subcore_barrier()

    def chunk(j, _):
        pltpu.async_copy(tab_hbm.at[src_v.at[j]], rows_v, sem).wait()
        pltpu.sync_copy(rows_v, acc.at[dst_v.at[j]], add=True)
        return 0
    lax.fori_loop(0, 80, chunk, 0)
    plsc.subcore_barrier()

    @pl.when(s < _NS - 1)
    def _():
        pltpu.sync_copy(acc.at[pl.ds(base, _RPT)],
                        out_hbm.at[c, pl.ds(base, _RPT)])

    @pl.when(s == _NS - 1)
    def _():
        pltpu.sync_copy(acc.at[pl.ds(base, _N - 15 * _RPT)],
                        out_hbm.at[c, pl.ds(base, _N - 15 * _RPT)])


_seg_call = functools.partial(
    pl.kernel,
    out_type=jax.ShapeDtypeStruct((_NC, _N, _H), jnp.float32),
    mesh=_MESH,
    scratch_types=[
        pltpu.VMEM((80, 128), jnp.int32),
        pltpu.VMEM((80, 128), jnp.int32),
        pltpu.VMEM((128, _H), jnp.float32),
        pltpu.VMEM_SHARED((_SROWS, _H), jnp.float32),
        pltpu.SemaphoreType.DMA,
    ],
    compiler_params=_SC_PARAMS,
)


def _dense_layer(segp_ref, xsp_ref, dinv_ref, M_ref, A_ref, bb_ref):
    seg = jnp.concatenate([segp_ref[0], segp_ref[1]], axis=-1)
    xs = jnp.concatenate([xsp_ref[0], xsp_ref[1]], axis=-1)
    y = (seg + xs) * dinv_ref[...]
    y = jnp.dot(y, M_ref[...], preferred_element_type=jnp.float32) + bb_ref[...]
    m = jnp.dot(y, A_ref[...], preferred_element_type=jnp.float32)
    d = y - m
    v = jnp.dot(d * d, A_ref[...], preferred_element_type=jnp.float32)
    return jnp.maximum(d * jax.lax.rsqrt(v + _EPS), 0.0)


def _layer_body(segp_ref, xsp_ref, dinv_ref, M_ref, A_ref, bb_ref, hsp_ref):
    h = _dense_layer(segp_ref, xsp_ref, dinv_ref, M_ref, A_ref, bb_ref)
    hs = h * dinv_ref[...]
    hsp_ref[0] = hs[:, :_H]
    hsp_ref[1] = hs[:, _H:]


_PACKED = pl.BlockSpec((_NC, _TN, _H), lambda i: (0, i, 0))
_ROWSCALE = pl.BlockSpec((_TN, 1), lambda i: (i, 0))
_FULL = pl.BlockSpec((_C, _C), lambda i: (0, 0))
_VEC = pl.BlockSpec((1, _C), lambda i: (0, 0))


def _layer_tc(segp, xsp, dinv, M, A, bb):
    return pl.pallas_call(
        _layer_body,
        grid=(_N // _TN,),
        in_specs=[_PACKED, _PACKED, _ROWSCALE, _FULL, _FULL, _VEC],
        out_specs=_PACKED,
        out_shape=jax.ShapeDtypeStruct((_NC, _N, _H), jnp.float32),
    )(segp, xsp, dinv, M, A, bb)


def _layer3_body(segp_ref, xsp_ref, dinv_ref, M_ref, A_ref, bb_ref, P_ref,
                 wmu_ref, wvar_ref, bmu_ref, bvar_ref,
                 mu_ref, lv_ref, accmu, acclv):
    i = pl.program_id(0)

    @pl.when(i == 0)
    def _():
        accmu[...] = jnp.zeros_like(accmu)
        acclv[...] = jnp.zeros_like(acclv)

    h = _dense_layer(segp_ref, xsp_ref, dinv_ref, M_ref, A_ref, bb_ref)
    hperm = jnp.dot(h, P_ref[...], preferred_element_type=jnp.float32)
    tdims = (((0,), (0,)), ((), ()))
    mu = accmu[...]
    lv = acclv[...]
    for k in range(9):
        hk = hperm[:, k * _B:(k + 1) * _B]
        mu = mu + lax.dot_general(hk, wmu_ref[:, k, :], tdims,
                                  preferred_element_type=jnp.float32)
        lv = lv + lax.dot_general(hk, wvar_ref[:, k, :], tdims,
                                  preferred_element_type=jnp.float32)
    accmu[...] = mu
    acclv[...] = lv

    @pl.when(i == _N // _TN - 1)
    def _():
        mu_ref[...] = accmu[...] + bmu_ref[...]
        lv_ref[...] = acclv[...] + bvar_ref[...]


def _layer3_tc(segp, xsp, dinv, M, A, bb, P, Wmu, bmu, Wvar, bvar):
    wspec = pl.BlockSpec((_TN, 9, _FEAT), lambda i: (i, 0, 0))
    bspec = pl.BlockSpec((1, _FEAT), lambda i: (0, 0))
    return pl.pallas_call(
        _layer3_body,
        grid=(_N // _TN,),
        in_specs=[_PACKED, _PACKED, _ROWSCALE, _FULL, _FULL, _VEC, _FULL,
                  wspec, wspec, bspec, bspec],
        out_specs=[pl.BlockSpec((_B, _FEAT), lambda i: (0, 0))] * 2,
        out_shape=[jax.ShapeDtypeStruct((_B, _FEAT), jnp.float32)] * 2,
        scratch_shapes=[pltpu.VMEM((_B, _FEAT), jnp.float32)] * 2,
    )(segp, xsp, dinv, M, A, bb, P,
      Wmu.reshape(_N, 9, _FEAT), Wvar.reshape(_N, 9, _FEAT),
      bmu.reshape(1, _FEAT), bvar.reshape(1, _FEAT))


def kernel(featurein, edge_index, W1, b1, W2, b2, W3, b3, Wmu, bmu, Wvar, bvar):
    src = edge_index[0]
    dst = edge_index[1]

    npad = _EP - _E
    srcp = jnp.concatenate([src, jnp.zeros((npad,), jnp.int32)])
    dstp = jnp.concatenate([dst, jnp.full((npad,), _TRASH, jnp.int32)])
    src_seg = (srcp[None, :] + jnp.array([0, _N], jnp.int32)[:, None]
               ).reshape(_NC, _NS, 80, 128)
    dst_seg = dstp.reshape(_NS, 80, 128)
    dst_deg = dstp.reshape(_NC * _NS, 40, 128)

    degp = _deg_call(_deg_body)(dst_deg)
    deg = degp[0, :_N, 0] + degp[1, :_N, 0] + 1.0
    dinv = jax.lax.rsqrt(deg).reshape(_N, 1)

    x_t = featurein.transpose(1, 0, 2).reshape(_N, _C)
    hsp = (x_t * dinv).reshape(_N, _NC, _H).transpose(1, 0, 2)

    eye = jnp.eye(_B, dtype=jnp.float32)
    A = jnp.kron(eye, jnp.full((9, 9), 1.0 / 9.0, jnp.float32))
    bidx = jnp.arange(_B, dtype=jnp.int32)
    kidx = jnp.arange(9, dtype=jnp.int32)
    rows = (bidx[:, None] * 9 + kidx[None, :]).ravel()
    cols = (kidx[None, :] * _B + bidx[:, None]).ravel()
    P = jnp.zeros((_C, _C), jnp.float32).at[rows, cols].set(1.0)

    seg_fn = _seg_call(_seg_body)
    Ms = [jnp.kron(eye, W) for W in (W1, W2, W3)]
    bbs = [jnp.tile(b, (_B,)).reshape(1, _C) for b in (b1, b2, b3)]
    for l in range(2):
        segp = seg_fn(hsp.reshape(_NC * _N, _H), src_seg, dst_seg)
        hsp = _layer_tc(segp, hsp, dinv, Ms[l], A, bbs[l])
    segp = seg_fn(hsp.reshape(_NC * _N, _H), src_seg, dst_seg)
    mu, logvar = _layer3_tc(segp, hsp, dinv, Ms[2], A, bbs[2], P,
                            Wmu, bmu, Wvar, bvar)
    return (mu, logvar)


# speedup vs baseline: 1.2677x; 1.0361x over previous
import functools

import jax
import jax.numpy as jnp
from jax import lax
from jax.experimental import pallas as pl
from jax.experimental.pallas import tpu as pltpu, tpu_sc as plsc

_B, _N, _E, _FEAT = 32, 10000, 160000, 256
_C = _B * 9
_H = _C // 2
_NC, _NS = 2, 16
_EP = 163840
_TRASH = _N
_DROWS = 10240
_SROWS = 10112
_RPT = 632
_TN = 400
_EPS = 1e-5

_MESH = plsc.VectorSubcoreMesh(
    core_axis_name="c", subcore_axis_name="s", num_cores=_NC, num_subcores=_NS)
_SC_PARAMS = pltpu.CompilerParams(use_tc_tiling_on_sc=False)


def _fill_rows(ref, nrows, ncol16, value):
    def row(i, _):
        def col(k, _):
            ref[i, pl.ds(k * 16, 16)] = jnp.full((16,), value, ref.dtype)
            return 0
        return lax.fori_loop(0, ncol16, col, 0)
    lax.fori_loop(0, nrows, row, 0)


def _deg_body(dst_hbm, out_hbm, dst_v, ones_v, stage_v, acc, sem):
    c = lax.axis_index("c")
    s = lax.axis_index("s")
    w = c * _NS + s
    pltpu.sync_copy(dst_hbm.at[w], dst_v)
    _fill_rows(ones_v, 128, 1, 1.0)
    _fill_rows(stage_v, 640, 1, 0.0)
    pltpu.sync_copy(stage_v, acc.at[pl.ds(s * 640, 640)])
    plsc.subcore_barrier()

    def chunk(j, _):
        pltpu.sync_copy(ones_v, acc.at[dst_v.at[j]], add=True)
        return 0
    lax.fori_loop(0, 40, chunk, 0)
    plsc.subcore_barrier()
    pltpu.sync_copy(acc.at[pl.ds(s * 640, 640)], stage_v)
    pltpu.sync_copy(stage_v, out_hbm.at[c, pl.ds(s * 640, 640)])


_deg_call = functools.partial(
    pl.kernel,
    out_type=jax.ShapeDtypeStruct((_NC, _DROWS, 16), jnp.float32),
    mesh=_MESH,
    scratch_types=[
        pltpu.VMEM((40, 128), jnp.int32),
        pltpu.VMEM((128, 16), jnp.float32),
        pltpu.VMEM((640, 16), jnp.float32),
        pltpu.VMEM_SHARED((_DROWS, 16), jnp.float32),
        pltpu.SemaphoreType.DMA,
    ],
    compiler_params=_SC_PARAMS,
)


def _seg_body(tab_hbm, src_hbm, dst_hbm, out_hbm, src_v, dst_v, rows_v, acc,
              *sems):
    c = lax.axis_index("c")
    s = lax.axis_index("s")
    sg, ss = sems[:2], sems[2:]
    pltpu.sync_copy(src_hbm.at[c, s], src_v)
    pltpu.sync_copy(dst_hbm.at[s], dst_v)
    _fill_rows(rows_v.at[0], 64, 9, 0.0)
    base = s * _RPT
    for p in range(9):
        pltpu.sync_copy(rows_v.at[0], acc.at[pl.ds(base + p * 64, 64)])
    pltpu.sync_copy(rows_v.at[0, pl.ds(0, _RPT - 576)],
                    acc.at[pl.ds(base + 576, _RPT - 576)])
    plsc.subcore_barrier()

    for b in range(2):
        pltpu.async_copy(tab_hbm.at[src_v.at[b]], rows_v.at[b], sg[b])

    def pair(g, _):
        j0 = g * 2
        for b in range(2):
            pltpu.make_async_copy(tab_hbm.at[src_v.at[j0 + b]],
                                  rows_v.at[b], sg[b]).wait()
            pltpu.async_copy(rows_v.at[b], acc.at[dst_v.at[j0 + b]],
                             ss[b], add=True)
        for b in range(2):
            jn = jnp.minimum(j0 + 2 + b, 159)
            pltpu.make_async_copy(rows_v.at[b], acc.at[dst_v.at[j0 + b]],
                                  ss[b]).wait()
            pltpu.async_copy(tab_hbm.at[src_v.at[jn]], rows_v.at[b], sg[b])
        return 0
    lax.fori_loop(0, 80, pair, 0)
    for b in range(2):
        pltpu.make_async_copy(tab_hbm.at[src_v.at[0]], rows_v.at[b],
                              sg[b]).wait()
    plsc.subcore_barrier()

    @pl.when(s < _NS - 1)
    def _():
        pltpu.sync_copy(acc.at[pl.ds(base, _RPT)],
                        out_hbm.at[c, pl.ds(base, _RPT)])

    @pl.when(s == _NS - 1)
    def _():
        pltpu.sync_copy(acc.at[pl.ds(base, _N - 15 * _RPT)],
                        out_hbm.at[c, pl.ds(base, _N - 15 * _RPT)])


_seg_call = functools.partial(
    pl.kernel,
    out_type=jax.ShapeDtypeStruct((_NC, _N, _H), jnp.float32),
    mesh=_MESH,
    scratch_types=[
        pltpu.VMEM((160, 64), jnp.int32),
        pltpu.VMEM((160, 64), jnp.int32),
        pltpu.VMEM((2, 64, _H), jnp.float32),
        pltpu.VMEM_SHARED((_SROWS, _H), jnp.float32),
    ] + [pltpu.SemaphoreType.DMA] * 4,
    compiler_params=_SC_PARAMS,
)


def _dense_layer(segp_ref, xsp_ref, dinv_ref, M_ref, A_ref, bb_ref):
    seg = jnp.concatenate([segp_ref[0], segp_ref[1]], axis=-1)
    xs = jnp.concatenate([xsp_ref[0], xsp_ref[1]], axis=-1)
    y = (seg + xs) * dinv_ref[...]
    y = jnp.dot(y, M_ref[...], preferred_element_type=jnp.float32) + bb_ref[...]
    m = jnp.dot(y, A_ref[...], preferred_element_type=jnp.float32)
    d = y - m
    v = jnp.dot(d * d, A_ref[...], preferred_element_type=jnp.float32)
    return jnp.maximum(d * jax.lax.rsqrt(v + _EPS), 0.0)


def _layer_body(segp_ref, xsp_ref, dinv_ref, M_ref, A_ref, bb_ref, hsp_ref):
    h = _dense_layer(segp_ref, xsp_ref, dinv_ref, M_ref, A_ref, bb_ref)
    hs = h * dinv_ref[...]
    hsp_ref[0] = hs[:, :_H]
    hsp_ref[1] = hs[:, _H:]


_PACKED = pl.BlockSpec((_NC, _TN, _H), lambda i: (0, i, 0))
_ROWSCALE = pl.BlockSpec((_TN, 1), lambda i: (i, 0))
_FULL = pl.BlockSpec((_C, _C), lambda i: (0, 0))
_VEC = pl.BlockSpec((1, _C), lambda i: (0, 0))


def _layer_tc(segp, xsp, dinv, M, A, bb):
    return pl.pallas_call(
        _layer_body,
        grid=(_N // _TN,),
        in_specs=[_PACKED, _PACKED, _ROWSCALE, _FULL, _FULL, _VEC],
        out_specs=_PACKED,
        out_shape=jax.ShapeDtypeStruct((_NC, _N, _H), jnp.float32),
    )(segp, xsp, dinv, M, A, bb)


def _layer3_body(segp_ref, xsp_ref, dinv_ref, M_ref, A_ref, bb_ref, P_ref,
                 wmu_ref, wvar_ref, bmu_ref, bvar_ref,
                 mu_ref, lv_ref, accmu, acclv):
    i = pl.program_id(0)

    @pl.when(i == 0)
    def _():
        accmu[...] = jnp.zeros_like(accmu)
        acclv[...] = jnp.zeros_like(acclv)

    h = _dense_layer(segp_ref, xsp_ref, dinv_ref, M_ref, A_ref, bb_ref)
    hperm = jnp.dot(h, P_ref[...], preferred_element_type=jnp.float32)
    tdims = (((0,), (0,)), ((), ()))
    mu = accmu[...]
    lv = acclv[...]
    for k in range(9):
        hk = hperm[:, k * _B:(k + 1) * _B]
        mu = mu + lax.dot_general(hk, wmu_ref[:, k, :], tdims,
                                  preferred_element_type=jnp.float32)
        lv = lv + lax.dot_general(hk, wvar_ref[:, k, :], tdims,
                                  preferred_element_type=jnp.float32)
    accmu[...] = mu
    acclv[...] = lv

    @pl.when(i == _N // _TN - 1)
    def _():
        mu_ref[...] = accmu[...] + bmu_ref[...]
        lv_ref[...] = acclv[...] + bvar_ref[...]


def _layer3_tc(segp, xsp, dinv, M, A, bb, P, Wmu, bmu, Wvar, bvar):
    wspec = pl.BlockSpec((_TN, 9, _FEAT), lambda i: (i, 0, 0))
    bspec = pl.BlockSpec((1, _FEAT), lambda i: (0, 0))
    return pl.pallas_call(
        _layer3_body,
        grid=(_N // _TN,),
        in_specs=[_PACKED, _PACKED, _ROWSCALE, _FULL, _FULL, _VEC, _FULL,
                  wspec, wspec, bspec, bspec],
        out_specs=[pl.BlockSpec((_B, _FEAT), lambda i: (0, 0))] * 2,
        out_shape=[jax.ShapeDtypeStruct((_B, _FEAT), jnp.float32)] * 2,
        scratch_shapes=[pltpu.VMEM((_B, _FEAT), jnp.float32)] * 2,
    )(segp, xsp, dinv, M, A, bb, P,
      Wmu.reshape(_N, 9, _FEAT), Wvar.reshape(_N, 9, _FEAT),
      bmu.reshape(1, _FEAT), bvar.reshape(1, _FEAT))


def kernel(featurein, edge_index, W1, b1, W2, b2, W3, b3, Wmu, bmu, Wvar, bvar):
    src = edge_index[0]
    dst = edge_index[1]

    npad = _EP - _E
    srcp = jnp.concatenate([src, jnp.zeros((npad,), jnp.int32)])
    dstp = jnp.concatenate([dst, jnp.full((npad,), _TRASH, jnp.int32)])
    src_seg = (srcp[None, :] + jnp.array([0, _N], jnp.int32)[:, None]
               ).reshape(_NC, _NS, 160, 64)
    dst_seg = dstp.reshape(_NS, 160, 64)
    dst_deg = dstp.reshape(_NC * _NS, 40, 128)

    degp = _deg_call(_deg_body)(dst_deg)
    deg = degp[0, :_N, 0] + degp[1, :_N, 0] + 1.0
    dinv = jax.lax.rsqrt(deg).reshape(_N, 1)

    x_t = featurein.transpose(1, 0, 2).reshape(_N, _C)
    hsp = (x_t * dinv).reshape(_N, _NC, _H).transpose(1, 0, 2)

    eye = jnp.eye(_B, dtype=jnp.float32)
    A = jnp.kron(eye, jnp.full((9, 9), 1.0 / 9.0, jnp.float32))
    bidx = jnp.arange(_B, dtype=jnp.int32)
    kidx = jnp.arange(9, dtype=jnp.int32)
    rows = (bidx[:, None] * 9 + kidx[None, :]).ravel()
    cols = (kidx[None, :] * _B + bidx[:, None]).ravel()
    P = jnp.zeros((_C, _C), jnp.float32).at[rows, cols].set(1.0)

    seg_fn = _seg_call(_seg_body)
    Ms = [jnp.kron(eye, W) for W in (W1, W2, W3)]
    bbs = [jnp.tile(b, (_B,)).reshape(1, _C) for b in (b1, b2, b3)]
    for l in range(2):
        segp = seg_fn(hsp.reshape(_NC * _N, _H), src_seg, dst_seg)
        hsp = _layer_tc(segp, hsp, dinv, Ms[l], A, bbs[l])
    segp = seg_fn(hsp.reshape(_NC * _N, _H), src_seg, dst_seg)
    mu, logvar = _layer3_tc(segp, hsp, dinv, Ms[2], A, bbs[2], P,
                            Wmu, bmu, Wvar, bvar)
    return (mu, logvar)
